# baseline (speedup 1.0000x reference)
import functools
import jax
import jax.numpy as jnp
from jax import lax
from jax.experimental import pallas as pl
from jax.experimental.pallas import tpu as pltpu
from jax.experimental.pallas import tpu_sc as plsc

E = 64
K = 8
R = 16
TEMP = 0.5
TILE = 2048
NWORKERS = 32


def _h_kernel(x_ref, w_ref, b_ref, h_ref, mx_ref, smx):
    i = pl.program_id(0)
    h = lax.dot_general(x_ref[...], w_ref[:E, :], (((1,), (1,)), ((), ())),
                        preferred_element_type=jnp.float32)
    h = h + b_ref[:, :E]
    h_ref[...] = h
    tmax = jnp.max(jnp.abs(h))

    @pl.when(i == 0)
    def _():
        smx[0] = tmax

    @pl.when(i != 0)
    def _():
        smx[0] = jnp.maximum(smx[0], tmax)

    mx_ref[...] = jnp.full((1, 16), smx[0], jnp.float32)


def _sc_routing(h, mx16, limes, n_tokens):
    chunk = n_tokens // NWORKERS
    groups = chunk // 16
    mesh = plsc.VectorSubcoreMesh(core_axis_name="c", subcore_axis_name="s")

    @functools.partial(
        pl.kernel,
        out_type=jax.ShapeDtypeStruct((n_tokens * R,), jnp.float32),
        mesh=mesh,
        scratch_types=[
            pltpu.VMEM((chunk * E,), jnp.float32),
            pltpu.VMEM((E * R,), jnp.float32),
            pltpu.VMEM((16,), jnp.float32),
            pltpu.VMEM((chunk * R,), jnp.float32),
        ],
        compiler_params=pltpu.CompilerParams(needs_layout_passes=False),
    )
    def body(h_hbm, mx_hbm, l_hbm, out_hbm, h_v, l_v, mx_v, p_v):
        wid = lax.axis_index("s") * 2 + lax.axis_index("c")
        pltpu.sync_copy(h_hbm.at[pl.ds(wid * chunk * E, chunk * E)], h_v)
        pltpu.sync_copy(l_hbm, l_v)
        pltpu.sync_copy(mx_hbm, mx_v)
        scale = jnp.maximum(mx_v[...], 1e-6)
        inv = (1.0 / TEMP) / scale
        iota16 = lax.iota(jnp.int32, 16)

        @plsc.parallel_loop(0, groups, unroll=4)
        def group(g):
            rowE = (g * 16 + iota16) * E
            top = [jnp.zeros((16,), jnp.float32)] * K
            for e in range(E):
                he = plsc.load_gather(h_v, [rowE + e])
                vbits = lax.bitcast_convert_type(he + scale, jnp.int32)
                new = lax.bitcast_convert_type(
                    (vbits & -64) | (63 - e), jnp.float32)
                for j in range(K):
                    hi = jnp.maximum(top[j], new)
                    new = jnp.minimum(top[j], new)
                    top[j] = hi
            ws = []
            eidx = []
            s = jnp.zeros((16,), jnp.float32)
            for j in range(K):
                tb = lax.bitcast_convert_type(top[j], jnp.int32)
                hv = lax.bitcast_convert_type(tb & -64, jnp.float32) - scale
                wv = jnp.exp(hv * inv)
                ws.append(wv)
                eidx.append((63 - (tb & 63)) * R)
                s = s + wv
            sinv = 1.0 / s
            ws = [wv * sinv for wv in ws]
            rowR = (g * 16 + iota16) * R
            for r in range(R):
                acc = jnp.zeros((16,), jnp.float32)
                for j in range(K):
                    lv = plsc.load_gather(l_v, [eidx[j] + r])
                    acc = acc + ws[j] * lv
                plsc.store_scatter(p_v, [rowR + r], acc)

        pltpu.sync_copy(p_v, out_hbm.at[pl.ds(wid * chunk * R, chunk * R)])

    return body(h.reshape(-1), mx16, limes.reshape(-1)).reshape(n_tokens, R)


def _main_kernel(x_ref, w_ref, b_ref, a_ref, bm_ref, p_ref, o_ref):
    x = x_ref[...]
    base = lax.dot_general(x, w_ref[...], (((1,), (1,)), ((), ())),
                           preferred_element_type=jnp.float32)
    base = base + b_ref[...]
    u = jnp.dot(x, a_ref[...], preferred_element_type=jnp.float32)
    delta = jnp.dot(u * p_ref[...], bm_ref[...],
                    preferred_element_type=jnp.float32)
    o_ref[...] = base + delta


def kernel(x, W, b, A, Bm, LiMEs):
    Bb, T, D_in = x.shape
    D_out = W.shape[0]
    N = Bb * T
    NT = N // TILE
    x2 = x.reshape(N, D_in)

    h, mx = pl.pallas_call(
        _h_kernel,
        grid=(NT,),
        in_specs=[
            pl.BlockSpec((TILE, D_in), lambda i: (i, 0)),
            pl.BlockSpec((D_out, D_in), lambda i: (0, 0)),
            pl.BlockSpec((1, D_out), lambda i: (0, 0)),
        ],
        out_specs=[
            pl.BlockSpec((TILE, E), lambda i: (i, 0)),
            pl.BlockSpec((1, 16), lambda i: (0, 0)),
        ],
        out_shape=[
            jax.ShapeDtypeStruct((N, E), jnp.float32),
            jax.ShapeDtypeStruct((1, 16), jnp.float32),
        ],
        scratch_shapes=[pltpu.SMEM((1,), jnp.float32)],
    )(x2, W, b.reshape(1, D_out))

    p_mix = _sc_routing(h, mx.reshape(16), LiMEs, N)

    out = pl.pallas_call(
        _main_kernel,
        grid=(NT,),
        in_specs=[
            pl.BlockSpec((TILE, D_in), lambda i: (i, 0)),
            pl.BlockSpec((D_out, D_in), lambda i: (0, 0)),
            pl.BlockSpec((1, D_out), lambda i: (0, 0)),
            pl.BlockSpec((D_in, R), lambda i: (0, 0)),
            pl.BlockSpec((R, D_out), lambda i: (0, 0)),
            pl.BlockSpec((TILE, R), lambda i: (i, 0)),
        ],
        out_specs=pl.BlockSpec((TILE, D_out), lambda i: (i, 0)),
        out_shape=jax.ShapeDtypeStruct((N, D_out), jnp.float32),
        compiler_params=pltpu.CompilerParams(
            vmem_limit_bytes=100 * 1024 * 1024),
    )(x2, W, b.reshape(1, D_out), A, Bm, p_mix)

    return out.reshape(Bb, T, D_out)


# speedup vs baseline: 1.0637x; 1.0637x over previous
import functools
import jax
import jax.numpy as jnp
from jax import lax
from jax.experimental import pallas as pl
from jax.experimental.pallas import tpu as pltpu
from jax.experimental.pallas import tpu_sc as plsc

E = 64
K = 8
R = 16
TEMP = 0.5
TILE = 2048
NWORKERS = 32


def _h_kernel(x_ref, w_ref, b_ref, h_ref, mx_ref, smx):
    i = pl.program_id(0)
    h = lax.dot_general(x_ref[...], w_ref[:E, :], (((1,), (1,)), ((), ())),
                        preferred_element_type=jnp.float32)
    h = h + b_ref[:, :E]
    h_ref[...] = h
    tmax = jnp.max(jnp.abs(h))

    @pl.when(i == 0)
    def _():
        smx[0] = tmax

    @pl.when(i != 0)
    def _():
        smx[0] = jnp.maximum(smx[0], tmax)

    mx_ref[...] = jnp.full((1, 16), smx[0], jnp.float32)


def _sc_routing(h, mx16, limes, n_tokens):
    chunk = n_tokens // NWORKERS
    groups = chunk // 16
    mesh = plsc.VectorSubcoreMesh(core_axis_name="c", subcore_axis_name="s")

    @functools.partial(
        pl.kernel,
        out_type=jax.ShapeDtypeStruct((n_tokens * R,), jnp.float32),
        mesh=mesh,
        scratch_types=[
            pltpu.VMEM((chunk * E,), jnp.float32),
            pltpu.VMEM((E * R,), jnp.float32),
            pltpu.VMEM((16,), jnp.float32),
            pltpu.VMEM((chunk * R,), jnp.float32),
        ],
        compiler_params=pltpu.CompilerParams(needs_layout_passes=False),
    )
    def body(h_hbm, mx_hbm, l_hbm, out_hbm, h_v, l_v, mx_v, p_v):
        wid = lax.axis_index("s") * 2 + lax.axis_index("c")
        pltpu.sync_copy(h_hbm.at[pl.ds(wid * chunk * E, chunk * E)], h_v)
        pltpu.sync_copy(l_hbm, l_v)
        pltpu.sync_copy(mx_hbm, mx_v)
        scale = jnp.maximum(mx_v[...], 1e-6)
        inv = (1.0 / TEMP) / scale
        iota16 = lax.iota(jnp.int32, 16)

        @plsc.parallel_loop(0, groups, unroll=2)
        def group(g):
            rowE = (g * 16 + iota16) * E
            top = [jnp.zeros((16,), jnp.float32)] * K
            for e in range(E):
                he = plsc.load_gather(h_v, [rowE + e])
                vbits = lax.bitcast_convert_type(he + scale, jnp.int32)
                new = lax.bitcast_convert_type(
                    (vbits & -64) | (63 - e), jnp.float32)
                for j in range(K):
                    hi = jnp.maximum(top[j], new)
                    new = jnp.minimum(top[j], new)
                    top[j] = hi
            ws = []
            eidx = []
            s = jnp.zeros((16,), jnp.float32)
            for j in range(K):
                tb = lax.bitcast_convert_type(top[j], jnp.int32)
                hv = lax.bitcast_convert_type(tb & -64, jnp.float32) - scale
                wv = jnp.exp(hv * inv)
                ws.append(wv)
                eidx.append((63 - (tb & 63)) * R)
                s = s + wv
            sinv = 1.0 / s
            ws = [wv * sinv for wv in ws]
            rowR = (g * 16 + iota16) * R
            for r in range(R):
                acc = jnp.zeros((16,), jnp.float32)
                for j in range(K):
                    lv = plsc.load_gather(l_v, [eidx[j] + r])
                    acc = acc + ws[j] * lv
                plsc.store_scatter(p_v, [rowR + r], acc)

        pltpu.sync_copy(p_v, out_hbm.at[pl.ds(wid * chunk * R, chunk * R)])

    return body(h.reshape(-1), mx16, limes.reshape(-1)).reshape(n_tokens, R)


def _main_kernel(x_ref, w_ref, b_ref, a_ref, bm_ref, p_ref, o_ref):
    x = x_ref[...]
    base = lax.dot_general(x, w_ref[...], (((1,), (1,)), ((), ())),
                           preferred_element_type=jnp.float32)
    base = base + b_ref[...]
    u = jnp.dot(x, a_ref[...], preferred_element_type=jnp.float32)
    delta = jnp.dot(u * p_ref[...], bm_ref[...],
                    preferred_element_type=jnp.float32)
    o_ref[...] = base + delta


def kernel(x, W, b, A, Bm, LiMEs):
    Bb, T, D_in = x.shape
    D_out = W.shape[0]
    N = Bb * T
    NT = N // TILE
    x2 = x.reshape(N, D_in)

    h, mx = pl.pallas_call(
        _h_kernel,
        grid=(NT,),
        in_specs=[
            pl.BlockSpec((TILE, D_in), lambda i: (i, 0)),
            pl.BlockSpec((D_out, D_in), lambda i: (0, 0)),
            pl.BlockSpec((1, D_out), lambda i: (0, 0)),
        ],
        out_specs=[
            pl.BlockSpec((TILE, E), lambda i: (i, 0)),
            pl.BlockSpec((1, 16), lambda i: (0, 0)),
        ],
        out_shape=[
            jax.ShapeDtypeStruct((N, E), jnp.float32),
            jax.ShapeDtypeStruct((1, 16), jnp.float32),
        ],
        scratch_shapes=[pltpu.SMEM((1,), jnp.float32)],
    )(x2, W, b.reshape(1, D_out))

    p_mix = _sc_routing(h, mx.reshape(16), LiMEs, N)

    out = pl.pallas_call(
        _main_kernel,
        grid=(NT,),
        in_specs=[
            pl.BlockSpec((TILE, D_in), lambda i: (i, 0)),
            pl.BlockSpec((D_out, D_in), lambda i: (0, 0)),
            pl.BlockSpec((1, D_out), lambda i: (0, 0)),
            pl.BlockSpec((D_in, R), lambda i: (0, 0)),
            pl.BlockSpec((R, D_out), lambda i: (0, 0)),
            pl.BlockSpec((TILE, R), lambda i: (i, 0)),
        ],
        out_specs=pl.BlockSpec((TILE, D_out), lambda i: (i, 0)),
        out_shape=jax.ShapeDtypeStruct((N, D_out), jnp.float32),
        compiler_params=pltpu.CompilerParams(
            vmem_limit_bytes=100 * 1024 * 1024),
    )(x2, W, b.reshape(1, D_out), A, Bm, p_mix)

    return out.reshape(Bb, T, D_out)
